# baseline (speedup 1.0000x reference)
import jax
import jax.numpy as jnp
from jax.experimental import pallas as pl
from jax.experimental.pallas import tpu as pltpu

B, N, D, R = 4, 2048, 32, 16
GSTEPS = 16
ROWS = N * N * R // 128
TROW = ROWS // GSTEPS


def _body(a_ref, out_ref):
    out_ref[...] = jnp.zeros_like(out_ref) + a_ref[0, 0]


@jax.jit
def kernel(e_old, A, W, b):
    a3 = A.reshape(ROWS, 128)
    e_probe = pl.pallas_call(
        _body,
        grid=(GSTEPS,),
        in_specs=[pl.BlockSpec((TROW, 128), lambda gi: (gi, 0))],
        out_specs=pl.BlockSpec((8, 128), lambda gi: (0, 0)),
        out_shape=jax.ShapeDtypeStruct((8, 128), jnp.float32),
    )(a3)
    e_new = e_old * 0.0 + e_probe[0, 0]
    return jnp.concatenate([e_old, e_new], axis=-1)


# speedup vs baseline: 2.6931x; 2.6931x over previous
import jax
import jax.numpy as jnp
from jax.experimental import pallas as pl
from jax.experimental.pallas import tpu as pltpu

B, N, D, R = 4, 2048, 32, 16
TI = 128
NBLK = N // TI
NCORE = 2
BPC = NBLK // NCORE
CJ = 128
NC = N // CJ
WROW = N * R


def _body(e_hbm, a_hbm, p_hbm, b_hbm, out_hbm,
          e_vm, p_vm, b_vm, abuf, obuf, asem, osem, ssem):
    core = jax.lax.axis_index("core")

    pltpu.make_async_copy(e_hbm, e_vm, ssem).start()
    pltpu.make_async_copy(e_hbm, e_vm, ssem).wait()
    pltpu.make_async_copy(p_hbm, p_vm, ssem).start()
    pltpu.make_async_copy(p_hbm, p_vm, ssem).wait()
    pltpu.make_async_copy(b_hbm, b_vm, ssem).start()
    pltpu.make_async_copy(b_hbm, b_vm, ssem).wait()

    def a_copy(st):
        gi = core * BPC + st
        return pltpu.make_async_copy(
            a_hbm.at[pl.ds(gi * TI, TI), :], abuf.at[st % 2], asem.at[st % 2])

    a_copy(0).start()
    for st in range(BPC):
        if st + 1 < BPC:
            a_copy(st + 1).start()
        a_copy(st).wait()
        accs = [jnp.zeros((TI, D), jnp.float32) for _ in range(B)]
        for c in range(NC):
            a_c = abuf[st % 2, :, c * CJ * R:(c + 1) * CJ * R]
            sp = jax.lax.dot_general(
                a_c, p_vm[...], (((1,), (0,)), ((), ())),
                preferred_element_type=jnp.float32)
            sp = sp + b_vm[...]
            s = jnp.where(sp >= 0, sp, 0.2 * sp)
            for bb in range(B):
                ej = e_vm[bb, c * CJ:(c + 1) * CJ, :]
                dots = jax.lax.dot_general(
                    e_vm[bb, pl.ds((core * BPC + st) * TI, TI), :], ej,
                    (((1,), (1,)), ((), ())),
                    preferred_element_type=jnp.float32)
                accs[bb] += jax.lax.dot_general(
                    dots * s, ej, (((1,), (0,)), ((), ())),
                    preferred_element_type=jnp.float32)
        for bb in range(B):
            obuf[st % 2, bb, :, :] = accs[bb]
        gi = core * BPC + st
        ocopy = pltpu.make_async_copy(
            obuf.at[st % 2], out_hbm.at[:, pl.ds(gi * TI, TI), :],
            osem.at[st % 2])
        ocopy.start()
        ocopy.wait()


@jax.jit
def kernel(e_old, A, W, b):
    inv_n = 1.0 / N
    p_mat = jnp.kron(jnp.eye(CJ, dtype=jnp.float32), (W[0] * inv_n)[:, None])
    b_row = jnp.broadcast_to(b * inv_n, (1, 1))
    a2 = A.reshape(N, WROW)

    mesh = pltpu.create_tensorcore_mesh("core", num_cores=NCORE)
    e_new = pl.kernel(
        _body,
        out_type=jax.ShapeDtypeStruct((B, N, D), jnp.float32),
        mesh=mesh,
        scratch_types=[
            pltpu.VMEM((B, N, D), jnp.float32),
            pltpu.VMEM((CJ * R, CJ), jnp.float32),
            pltpu.VMEM((1, 1), jnp.float32),
            pltpu.VMEM((2, TI, WROW), jnp.float32),
            pltpu.VMEM((2, B, TI, D), jnp.float32),
            pltpu.SemaphoreType.DMA((2,)),
            pltpu.SemaphoreType.DMA((2,)),
            pltpu.SemaphoreType.DMA,
        ],
        compiler_params=pltpu.CompilerParams(
            vmem_limit_bytes=100 * 1024 * 1024),
    )(e_old, a2, p_mat, b_row)

    return jnp.concatenate([e_old, e_new], axis=-1)
